# baseline (speedup 1.0000x reference)
import functools

import jax
import jax.numpy as jnp
from jax import lax
from jax.experimental import pallas as pl
from jax.experimental.pallas import tpu as pltpu
from jax.experimental.pallas import tpu_sc as plsc

V = 1000000
D = 64
BATCH = 1024
S = 200
N = BATCH * S
NW = 32
TPW = N // NW
TAIL0 = 999936
ACAP = TPW + 8 * NW
IDSB_LEN = NW * ACAP + 2048
CAP = 16384
LCAP = CAP + 512
BLK = 512
DUMP = N
CBLK = 256
UPT = (S * BATCH // CBLK) // NW

_MESH = dict(core_axis_name="c", subcore_axis_name="s")


def _al8(x):
    return pl.multiple_of(x, 8)


def _wid():
    return lax.axis_index("s") * 2 + lax.axis_index("c")


def _bucket_of(v):
    b = jnp.minimum(v >> 15, 30)
    return jnp.where(v >= TAIL0, jnp.full_like(v, 31), b)


def _sort_rank(b_vec, iota16):
    bs, perm = plsc.sort_key_val(b_vec, iota16)
    prev = bs.at[jnp.maximum(iota16 - 1, 0)].get(mode="promise_in_bounds")
    isst = (iota16 == 0) | (bs != prev)
    spos = plsc.cummax(jnp.where(isst, iota16, jnp.zeros_like(iota16)))
    rank = iota16 - spos
    return bs, perm, rank


def _aligned_excl(h):
    a = (h + 7) & -8
    c = plsc.cumsum(a)
    return c - a, c[15]


def _phase_a():
    mesh = plsc.VectorSubcoreMesh(**_MESH)

    @functools.partial(
        pl.kernel, mesh=mesh,
        compiler_params=pltpu.CompilerParams(
            use_tc_tiling_on_sc=False, needs_layout_passes=False),
        out_type=(jax.ShapeDtypeStruct((IDSB_LEN,), jnp.int32),
                  jax.ShapeDtypeStruct((IDSB_LEN,), jnp.int32),
                  jax.ShapeDtypeStruct((1024,), jnp.int32)),
        scratch_types=[
            pltpu.VMEM((TPW,), jnp.int32),
            pltpu.VMEM((ACAP,), jnp.int32),
            pltpu.VMEM((ACAP,), jnp.int32),
            pltpu.VMEM((32,), jnp.int32),
            pltpu.VMEM((32,), jnp.int32),
        ],
    )
    def k(ids_hbm, idsb, posb, cnts, idv, sid, spo, hist, cur):
        w = _wid()
        iota16 = lax.iota(jnp.int32, 16)
        zeros = jnp.zeros((16,), jnp.int32)
        ones = jnp.ones((16,), jnp.int32)
        pltpu.sync_copy(ids_hbm.at[pl.ds(_al8(w * TPW), TPW)], idv)
        hist[pl.ds(0, 16)] = zeros
        hist[pl.ds(16, 16)] = zeros

        def h_body(k2, carry):
            v = idv[pl.ds(_al8(k2 * 16), 16)]
            plsc.addupdate_scatter(hist, [_bucket_of(v)], ones)
            return carry

        lax.fori_loop(0, TPW // 16, h_body, 0)

        e0, t0 = _aligned_excl(hist[pl.ds(0, 16)])
        e1, _ = _aligned_excl(hist[pl.ds(16, 16)])
        cur[pl.ds(0, 16)] = e0
        cur[pl.ds(16, 16)] = e1 + t0

        def s_body(k2, carry):
            v = idv[pl.ds(_al8(k2 * 16), 16)]
            p = w * TPW + k2 * 16 + iota16
            bs, perm, rank = _sort_rank(_bucket_of(v), iota16)
            vs = v.at[perm].get(mode="promise_in_bounds")
            ps = p.at[perm].get(mode="promise_in_bounds")
            off = plsc.load_gather(cur, [bs]) + rank
            plsc.store_scatter(sid, [off], vs)
            plsc.store_scatter(spo, [off], ps)
            plsc.addupdate_scatter(cur, [bs], ones)
            return carry

        lax.fori_loop(0, TPW // 16, s_body, 0)

        pltpu.sync_copy(sid, idsb.at[pl.ds(_al8(w * ACAP), ACAP)])
        pltpu.sync_copy(spo, posb.at[pl.ds(_al8(w * ACAP), ACAP)])
        pltpu.sync_copy(hist, cnts.at[pl.ds(_al8(w * 32), 32)])

    return k


def _phase_b():
    mesh = plsc.VectorSubcoreMesh(**_MESH)

    @functools.partial(
        pl.kernel, mesh=mesh,
        compiler_params=pltpu.CompilerParams(
            use_tc_tiling_on_sc=True, needs_layout_passes=False),
        out_type=jax.ShapeDtypeStruct((N + 8, 128), jnp.float32),
        scratch_types=[
            pltpu.VMEM((64, BLK), jnp.float32),
            pltpu.VMEM((64, BLK), jnp.float32),
            pltpu.VMEM((64, 128), jnp.float32),
            pltpu.VMEM((LCAP,), jnp.int32),
            pltpu.VMEM((LCAP,), jnp.int32),
            pltpu.VMEM((2048,), jnp.int32),
            pltpu.VMEM((2048,), jnp.int32),
            pltpu.VMEM((16, 128), jnp.float32),
            pltpu.VMEM((16, 128), jnp.float32),
            pltpu.VMEM((1024,), jnp.int32),
            pltpu.VMEM((80,), jnp.int32),
            pltpu.VMEM((80,), jnp.int32),
            pltpu.VMEM((80,), jnp.int32),
            pltpu.VMEM((32,), jnp.int32),
            pltpu.VMEM((32,), jnp.int32),
            pltpu.SemaphoreType.DMA,
            pltpu.SemaphoreType.DMA,
            pltpu.SemaphoreType.DMA,
            pltpu.SemaphoreType.DMA,
        ],
    )
    def k(tab, tailpad, idsb, posb, cnts, rows, bb0, bb1, tailb, lid, lpo,
          chi, chp, st0, st1, cntv, hist, boff, cur, startv, cntw,
          bsem0, bsem1, ssem0, ssem1):
        r = _wid()
        iota16 = lax.iota(jnp.int32, 16)
        zeros = jnp.zeros((16,), jnp.int32)
        ones = jnp.ones((16,), jnp.int32)
        pltpu.sync_copy(cnts, cntv)
        base_r = jnp.where(r == 31, jnp.int32(TAIL0), r << 15)
        rv = jnp.zeros((16,), jnp.int32) + r

        for q in range(2):
            wb = (q * 16 + iota16) * 32
            acc = zeros
            for rp in range(31):
                g = plsc.load_gather(cntv, [wb + rp])
                acc = acc + jnp.where(rp < r, (g + 7) & -8, zeros)
            startv[pl.ds(q * 16, 16)] = acc
            cntw[pl.ds(q * 16, 16)] = plsc.load_gather(cntv, [wb + r])

        for q in range(5):
            hist[pl.ds(q * 16, 16)] = zeros

        def scan(with_pos, seg_fn):
            def w_body(w, c0):
                start = plsc.load_gather(startv, [zeros + w])[0]
                cnt = plsc.load_gather(cntw, [zeros + w])[0]
                nch = (cnt + 2047) >> 11

                def ch_body(ch, carry):
                    sbase = _al8(w * ACAP + start + ch * 2048)
                    pltpu.sync_copy(idsb.at[pl.ds(sbase, 2048)], chi)
                    if with_pos:
                        pltpu.sync_copy(posb.at[pl.ds(sbase, 2048)], chp)
                    nv = (jnp.minimum(cnt - ch * 2048, 2048) + 15) >> 4

                    def v_body(k2, c2):
                        m = iota16 < (cnt - ch * 2048 - k2 * 16)
                        v = chi[pl.ds(_al8(k2 * 16), 16)]
                        blk = jnp.clip((v - base_r) >> 9, 0, 63)
                        blkm = jnp.where(m, blk, jnp.full_like(blk, 64))
                        seg_fn(k2, blkm)
                        return c2

                    lax.fori_loop(0, nv, v_body, 0)
                    return carry

                lax.fori_loop(0, nch, ch_body, 0)
                return c0

            lax.fori_loop(0, 32, w_body, 0)

        def s1_body(k2, blkm):
            plsc.addupdate_scatter(hist, [blkm], ones)

        def s2_body(k2, blkm):
            v = chi[pl.ds(_al8(k2 * 16), 16)]
            p = chp[pl.ds(_al8(k2 * 16), 16)]
            bs, perm, rank = _sort_rank(blkm, iota16)
            vs = v.at[perm].get(mode="promise_in_bounds")
            ps = p.at[perm].get(mode="promise_in_bounds")
            off = plsc.load_gather(cur, [bs]) + rank
            off = jnp.minimum(off, LCAP - 1)
            plsc.store_scatter(lid, [off], vs)
            plsc.store_scatter(lpo, [off], ps)
            plsc.addupdate_scatter(cur, [bs], ones)

        scan(False, s1_body)

        carry = jnp.int32(0)
        for q in range(5):
            hq = hist[pl.ds(q * 16, 16)]
            aq = (hq + 7) & -8
            cq = plsc.cumsum(aq)
            boff[pl.ds(q * 16, 16)] = cq - aq + carry
            cur[pl.ds(q * 16, 16)] = cq - aq + carry
            carry = carry + cq[15]

        scan(True, s2_body)

        stags = (st0, st1)
        ssems = (ssem0, ssem1)

        def drain_stag(qq):
            pltpu.make_async_copy(stags[qq], rows.at[pl.ds(0, 16)],
                                  ssems[qq]).wait()

        def extract16(buf, cmax, blkbase, lo, cnt, k2, qq):
            m = iota16 < (cnt - k2 * 16)
            v = lid[pl.ds(_al8(lo + k2 * 16), 16)]
            p = lpo[pl.ds(_al8(lo + k2 * 16), 16)]
            cl = jnp.clip(v - blkbase, 0, cmax)
            dv = jnp.where(m, p, jnp.full_like(p, DUMP))
            stag = stags[qq]

            def d_body(dq, c3):
                for u in range(4):
                    dvv = zeros + (dq * 4 + u)
                    x = plsc.load_gather(buf, [dvv, cl])
                    plsc.store_scatter(stag, [iota16, dvv], x)
                return c3

            lax.fori_loop(0, 16, d_body, 0)
            pltpu.async_copy(stag, rows.at[dv], ssems[qq])

        def process(buf, cmax, blkbase, lo, cnt):
            npair = (cnt + 31) >> 5

            def p_body(i, c2):
                for qq in range(2):
                    @pl.when(i >= 1)
                    def _(qq=qq):
                        drain_stag(qq)
                    extract16(buf, cmax, blkbase, lo, cnt, i * 2 + qq, qq)
                return c2

            lax.fori_loop(0, npair, p_body, 0)

            @pl.when(npair >= 1)
            def _():
                drain_stag(0)
                drain_stag(1)

        nfull = jnp.where(r < 30, 64, jnp.where(r == 30, 33, 0))
        bbs = (bb0, bb1)
        bsems = (bsem0, bsem1)

        def stage_block(b, qq):
            pltpu.async_copy(tab.at[:, pl.ds(pl.multiple_of(base_r + b * BLK, 128), BLK)],
                             bbs[qq], bsems[qq])

        def wait_block(qq):
            pltpu.make_async_copy(tab.at[:, pl.ds(0, BLK)], bbs[qq],
                                  bsems[qq]).wait()

        @pl.when(nfull > 0)
        def _():
            stage_block(0, 0)

        def b_body(b, c2):
            for qq in range(2):
                @pl.when((b & 1) == qq)
                def _(qq=qq):
                    wait_block(qq)

                    @pl.when(b + 1 < nfull)
                    def _():
                        stage_block(b + 1, 1 - qq)
                    bv = jnp.zeros((16,), jnp.int32) + b
                    lo = plsc.load_gather(boff, [bv])[0]
                    cb = plsc.load_gather(hist, [bv])[0]
                    process(bbs[qq], BLK - 1, base_r + b * BLK, lo, cb)
            return c2

        lax.fori_loop(0, nfull, b_body, 0)

        @pl.when(r == 31)
        def _():
            pltpu.sync_copy(tailpad, tailb)
            lo = plsc.load_gather(boff, [zeros])[0]
            cb = plsc.load_gather(hist, [zeros])[0]
            process(tailb, 127, TAIL0, lo, cb)

    return k


def _phase_c():
    mesh = plsc.VectorSubcoreMesh(**_MESH)

    @functools.partial(
        pl.kernel, mesh=mesh,
        compiler_params=pltpu.CompilerParams(
            use_tc_tiling_on_sc=True, needs_layout_passes=False),
        out_type=jax.ShapeDtypeStruct((S, D, BATCH), jnp.float32),
        scratch_types=[
            pltpu.VMEM((CBLK, 128), jnp.float32),
            pltpu.VMEM((CBLK, 128), jnp.float32),
            pltpu.VMEM((D, CBLK), jnp.float32),
            pltpu.VMEM((CBLK,), jnp.int32),
            pltpu.VMEM((CBLK,), jnp.int32),
            pltpu.VMEM((64,), jnp.float32),
            pltpu.SemaphoreType.DMA,
            pltpu.SemaphoreType.DMA,
        ],
    )
    def k(rows, pe, out3, gb0, gb1, ob, ix0, ix1, pb, gsem0, gsem1):
        wid = _wid()
        iota16 = lax.iota(jnp.int32, 16)
        gbs = (gb0, gb1)
        ixs = (ix0, ix1)
        gsems = (gsem0, gsem1)

        def start_gather(u, qq):
            s = u >> 2
            qb = u & 3

            def i_body(t, c2):
                ixs[qq][pl.ds(_al8(t * 16), 16)] = \
                    (qb * CBLK + t * 16 + iota16) * S + s
                return c2

            lax.fori_loop(0, CBLK // 16, i_body, 0)
            pltpu.async_copy(rows.at[ixs[qq]], gbs[qq], gsems[qq])

        def wait_gather(qq):
            pltpu.make_async_copy(rows.at[pl.ds(0, CBLK)], gbs[qq],
                                  gsems[qq]).wait()

        start_gather(wid, 0)

        def u_body(kk, c2):
            u = wid + kk * NW
            s = u >> 2
            qb = u & 3
            for qq in range(2):
                @pl.when((kk & 1) == qq)
                def _(qq=qq):
                    wait_gather(qq)

                    @pl.when(kk + 1 < UPT)
                    def _():
                        start_gather(u + NW, 1 - qq)
                    pltpu.sync_copy(pe.at[pl.ds(_al8(s * 64), 64)], pb)
                    pvs = [pb[pl.ds(g * 16, 16)] for g in range(4)]

                    def jg_body(jg, c3):
                        jvec = jg * 16 + iota16
                        col = _al8(jg * 16)
                        for d in range(64):
                            x = plsc.load_gather(
                                gbs[qq], [jvec, jnp.zeros((16,),
                                                          jnp.int32) + d])
                            x = x + pvs[d // 16][d % 16]
                            ob[d, pl.ds(col, 16)] = x
                        return c3

                    lax.fori_loop(0, CBLK // 16, jg_body, 0)
                    pltpu.sync_copy(
                        ob, out3.at[s].at[:, pl.ds(pl.multiple_of(qb * CBLK, 128), CBLK)])
            return c2

        lax.fori_loop(0, UPT, u_body, 0)

    return k


def kernel(input_ids, table, pos_enc):
    ids_flat = input_ids.reshape(-1).astype(jnp.int32)
    tab_t = jnp.transpose(table)
    pe_flat = pos_enc.reshape(-1)
    tail_pad = jnp.pad(tab_t[:, TAIL0:], ((0, 0), (0, 128 - (V - TAIL0))))
    idsb, posb, cnts = _phase_a()(ids_flat)
    rows = _phase_b()(tab_t, tail_pad, idsb, posb, cnts)
    out3 = _phase_c()(rows, pe_flat)
    return jnp.transpose(out3, (2, 0, 1))


# speedup vs baseline: 3.1819x; 3.1819x over previous
import functools

import jax
import jax.numpy as jnp
from jax import lax
from jax.experimental import pallas as pl
from jax.experimental.pallas import tpu as pltpu
from jax.experimental.pallas import tpu_sc as plsc

DIM = 64
SEQ = 200
BATCH = 1024
NW = 32
SEQ_PER_W = BATCH // NW
SPLIT = 128
LANES = 16
NBUF = 4
DIMP = 128


def _make_kernel():
    mesh = plsc.VectorSubcoreMesh(core_axis_name="c", subcore_axis_name="s")

    scratch = (
        [pltpu.VMEM((SEQ,), jnp.int32) for _ in range(NBUF)]
        + [pltpu.VMEM((SEQ, DIMP), jnp.float32) for _ in range(NBUF)]
        + [pltpu.VMEM((SEQ, DIMP), jnp.float32)]
        + [pltpu.SemaphoreType.DMA for _ in range(2 * NBUF)]
    )

    @functools.partial(
        pl.kernel,
        mesh=mesh,
        compiler_params=pltpu.CompilerParams(
            use_tc_tiling_on_sc=True, needs_layout_passes=False),
        out_type=jax.ShapeDtypeStruct((BATCH * SEQ, DIMP), jnp.float32),
        scratch_types=scratch,
    )
    def k(ids_hbm, table_hbm, pe_hbm, out_hbm, *refs):
        idxs = refs[0:NBUF]
        bufs = refs[NBUF:2 * NBUF]
        pe_v = refs[2 * NBUF]
        gsems = refs[2 * NBUF + 1: 2 * NBUF + 1 + NBUF]
        osems = refs[2 * NBUF + 1 + NBUF: 2 * NBUF + 1 + 2 * NBUF]

        wid = lax.axis_index("s") * 2 + lax.axis_index("c")
        base_row = wid * SEQ_PER_W * SEQ
        pltpu.sync_copy(pe_hbm, pe_v)

        def start_gather(t, b):
            base = pl.multiple_of(base_row + t * SEQ, 8)
            pltpu.sync_copy(ids_hbm.at[pl.ds(base, SEQ)], idxs[b])
            pltpu.async_copy(table_hbm.at[idxs[b].at[pl.ds(0, SPLIT)]],
                             bufs[b].at[pl.ds(0, SPLIT)], gsems[b])
            pltpu.async_copy(table_hbm.at[idxs[b].at[pl.ds(SPLIT, SEQ - SPLIT)]],
                             bufs[b].at[pl.ds(SPLIT, SEQ - SPLIT)], gsems[b])

        def wait_gather(b):
            pltpu.make_async_copy(table_hbm.at[pl.ds(0, SEQ)], bufs[b],
                                  gsems[b]).wait()

        def add_pe(b):
            buf = bufs[b]

            @plsc.parallel_loop(0, SEQ, unroll=8)
            def _(r):
                for j in range(DIM // LANES):
                    plsc.addupdate(buf.at[r, pl.ds(j * LANES, LANES)],
                                   pe_v[r, pl.ds(j * LANES, LANES)])

        def process(t, b, regather):
            wait_gather(b)
            add_pe(b)
            cp = pltpu.async_copy(
                bufs[b],
                out_hbm.at[pl.ds(pl.multiple_of(base_row + t * SEQ, 8), SEQ)],
                osems[b])
            if regather:
                cp.wait()
                start_gather(t + NBUF, b)
            return cp

        for b in range(NBUF):
            start_gather(b, b)

        @pl.loop(0, SEQ_PER_W - NBUF, step=NBUF)
        def _(g):
            for b in range(NBUF):
                process(g + b, b, regather=True)

        tail = []
        for b in range(NBUF):
            tail.append(process(SEQ_PER_W - NBUF + b, b, regather=False))
        for cp in tail:
            cp.wait()

    return k


def kernel(input_ids, table, pos_enc):
    ids_flat = input_ids.reshape(-1).astype(jnp.int32)
    tabp = jnp.pad(table, ((0, 0), (0, DIMP - DIM)))
    pe = jnp.pad(pos_enc.reshape(SEQ, DIM), ((0, 0), (0, DIMP - DIM)))
    out = _make_kernel()(ids_flat, tabp, pe)
    return out[:, :DIM].reshape(BATCH, SEQ, DIM)
